# Initial kernel scaffold; baseline (speedup 1.0000x reference)
import jax
import jax.numpy as jnp
from jax.experimental import pallas as pl


def kernel(x, edge_index_1, edge_index_2, edge_index_3, batch, params):
    raise NotImplementedError("write your pallas kernel here")


# speedup vs baseline: 9.9336x; 9.9336x over previous
import functools

import jax
import jax.numpy as jnp
from jax import lax
from jax.experimental import pallas as pl
from jax.experimental.pallas import tpu as pltpu
from jax.experimental.pallas import tpu_sc as plsc

N_NODES = 10000
E_EDGES = 160000
DIM = 64
NGRAPH = 64

NC = 2
NS = 16
NW = NC * NS

EPT = E_EDGES // NW
CMIN = 125
NCH = EPT // CMIN
NPAD = NW * 4 * 80

_mesh = plsc.VectorSubcoreMesh(
    core_axis_name="c", subcore_axis_name="s", num_cores=NC, num_subcores=NS)


@functools.partial(
    pl.kernel,
    out_type=jax.ShapeDtypeStruct((3, NW, 4, 80, DIM), jnp.float32),
    mesh=_mesh,
    compiler_params=pltpu.CompilerParams(use_tc_tiling_on_sc=False),
    scratch_types=[
        pltpu.VMEM((4, 80), jnp.int32),
        pltpu.VMEM((80, DIM), jnp.float32),
        pltpu.SemaphoreType.DMA,
    ],
)
def _gather_nodes(x_hbm, w1_hbm, w2_hbm, w3_hbm, out_hbm, xs_v, rows_v, sem):
    c = lax.axis_index("c")
    s = lax.axis_index("s")
    wid = s * NC + c
    pltpu.sync_copy(x_hbm.at[wid], xs_v)
    for j, w in enumerate((w1_hbm, w2_hbm, w3_hbm)):
        for k in range(4):
            pltpu.async_copy(w.at[xs_v.at[k]], rows_v, sem).wait()
            pltpu.sync_copy(rows_v, out_hbm.at[j, wid, k])


@functools.partial(
    pl.kernel,
    out_type=jax.ShapeDtypeStruct((NC, N_NODES, DIM), jnp.float32),
    mesh=_mesh,
    compiler_params=pltpu.CompilerParams(use_tc_tiling_on_sc=False),
    scratch_types=[
        pltpu.VMEM((NCH, CMIN), jnp.int32),
        pltpu.VMEM((NCH, CMIN), jnp.int32),
        pltpu.VMEM((CMIN, DIM), jnp.float32),
        pltpu.VMEM((CMIN, DIM), jnp.float32),
        pltpu.VMEM_SHARED((N_NODES, DIM), jnp.float32),
        pltpu.SemaphoreType.DMA,
    ],
)
def _agg(h_hbm, src_hbm, dst_hbm, out_hbm, src_v, dst_v, rows_v, zbuf_v,
         acc_sh, sem):
    c = lax.axis_index("c")
    s = lax.axis_index("s")
    wid = s * NC + c
    rows_per_sub = N_NODES // NS

    zero = jnp.zeros((16,), jnp.float32)

    def _zero_row(i, carry):
        for cc in range(DIM // 16):
            zbuf_v[i, pl.ds(cc * 16, 16)] = zero
        return carry

    lax.fori_loop(0, CMIN, _zero_row, 0)
    base = s * rows_per_sub
    for k in range(rows_per_sub // CMIN):
        pltpu.sync_copy(zbuf_v, acc_sh.at[pl.ds(base + k * CMIN, CMIN)])
    plsc.subcore_barrier()

    pltpu.sync_copy(src_hbm.at[wid], src_v)
    pltpu.sync_copy(dst_hbm.at[wid], dst_v)

    def _chunk(ch, carry):
        pltpu.async_copy(h_hbm.at[src_v.at[ch]], rows_v, sem).wait()
        pltpu.sync_copy(rows_v, acc_sh.at[dst_v.at[ch]], add=True)
        return carry

    lax.fori_loop(0, NCH, _chunk, 0)
    plsc.subcore_barrier()
    pltpu.sync_copy(acc_sh.at[pl.ds(base, rows_per_sub)],
                    out_hbm.at[c, pl.ds(base, rows_per_sub)])


def _dot(a, b):
    return jnp.dot(a, b, preferred_element_type=jnp.float32)


def _bn_tail(cat, mW1, mb1, mW2, mb2, gam, bet, out):
    h = jnp.maximum(_dot(cat, mW1[...]) + mb1[...], 0.0)
    h = _dot(h, mW2[...]) + mb2[...]
    m = jnp.mean(h, axis=0, keepdims=True)
    v = jnp.mean((h - m) * (h - m), axis=0, keepdims=True)
    out[...] = gam[...] * (h - m) * lax.rsqrt(v + 1e-5) + bet[...]


def _layer1_body(g1, g2, g3, pa, pb, pc,
                 e1, b11, W21, b21, e2, b12, W22, b22, e3, b13, W23, b23,
                 mW1, mb1, mW2, mb2, gam, bet, out):
    xs = []
    for g, pp, eps, b1, W2, b2 in (
            (g1, pa, e1, b11, W21, b21),
            (g2, pb, e2, b12, W22, b22),
            (g3, pc, e3, b13, W23, b23)):
        t = jnp.maximum(
            (1.0 + eps[0, 0]) * g[...] + pp[0] + pp[1] + b1[...], 0.0)
        xs.append(jnp.maximum(_dot(t, W2[...]) + b2[...], 0.0))
    _bn_tail(jnp.concatenate(xs, axis=1), mW1, mb1, mW2, mb2, gam, bet, out)


def _layer_body(h, pa, pb, pc,
                e1, W11, b11, W21, b21, e2, W12, b12, W22, b22,
                e3, W13, b13, W23, b23,
                mW1, mb1, mW2, mb2, gam, bet, out):
    xs = []
    for pp, eps, W1, b1, W2, b2 in (
            (pa, e1, W11, b11, W21, b21),
            (pb, e2, W12, b12, W22, b22),
            (pc, e3, W13, b13, W23, b23)):
        hin = (1.0 + eps[0, 0]) * h[...] + pp[0] + pp[1]
        t = jnp.maximum(_dot(hin, W1[...]) + b1[...], 0.0)
        xs.append(jnp.maximum(_dot(t, W2[...]) + b2[...], 0.0))
    _bn_tail(jnp.concatenate(xs, axis=1), mW1, mb1, mW2, mb2, gam, bet, out)


def _final_body(r1, r2, r3, r4, bt,
                f1W, f1b, f2W, f2b, f3W, f3b, f4W, f4b, out):
    sel = lax.broadcasted_iota(jnp.int32, (NGRAPH, N_NODES), 0)
    P = (sel == bt[...]).astype(jnp.float32)
    counts = jnp.sum(P, axis=1, keepdims=True)
    hcat = jnp.concatenate([r1[...], r2[...], r3[...], r4[...]], axis=1)
    pooled = _dot(P, hcat) / jnp.maximum(counts, 1.0)
    h = jnp.maximum(_dot(pooled, f1W[...]) + f1b[...], 0.0)
    h = jnp.maximum(_dot(h, f2W[...]) + f2b[...], 0.0)
    h = jnp.maximum(_dot(h, f3W[...]) + f3b[...], 0.0)
    out[...] = _dot(h, f4W[...]) + f4b[...]


def _tc_call(body, out_shape, *args):
    return pl.pallas_call(
        body, out_shape=jax.ShapeDtypeStruct(out_shape, jnp.float32))(*args)


def _row(v):
    return v.reshape(1, -1)


def kernel(x, edge_index_1, edge_index_2, edge_index_3, batch, params):
    x_pad = jnp.concatenate(
        [x, jnp.zeros((NPAD - N_NODES,), jnp.int32)]).reshape(NW, 4, 80)
    srcs, dsts = [], []
    for e in (edge_index_1, edge_index_2, edge_index_3):
        srcs.append(e[0].reshape(NW, NCH, CMIN))
        dsts.append(e[1].reshape(NW, NCH, CMIN))

    g = _gather_nodes(x_pad, params['conv1_1']['W1'], params['conv1_2']['W1'],
                      params['conv1_3']['W1'])
    g = g.reshape(3, NPAD, DIM)[:, :N_NODES]
    gs = [g[0], g[1], g[2]]

    parts = [_agg(gs[j], srcs[j], dsts[j]) for j in range(3)]
    l1args = []
    for j in range(3):
        q = params['conv1_%d' % (j + 1)]
        l1args += [q['eps'].reshape(1, 1), _row(q['b1']), q['W2'],
                   _row(q['b2'])]
    q = params['mlp_1']
    bnq = params['bn_1']
    h = _tc_call(_layer1_body, (N_NODES, DIM), *gs, *parts, *l1args,
                 q['W1'], _row(q['b1']), q['W2'], _row(q['b2']),
                 _row(bnq['gamma']), _row(bnq['beta']))
    reps = [h]

    for l in range(2, 5):
        parts = [_agg(h, srcs[j], dsts[j]) for j in range(3)]
        largs = []
        for j in range(3):
            q = params['conv%d_%d' % (l, j + 1)]
            largs += [q['eps'].reshape(1, 1), q['W1'], _row(q['b1']),
                      q['W2'], _row(q['b2'])]
        q = params['mlp_%d' % l]
        bnq = params['bn_%d' % l]
        h = _tc_call(_layer_body, (N_NODES, DIM), h, *parts, *largs,
                     q['W1'], _row(q['b1']), q['W2'], _row(q['b2']),
                     _row(bnq['gamma']), _row(bnq['beta']))
        reps.append(h)

    f4W = jnp.pad(params['fc4']['W'], ((0, 0), (0, 7)))
    f4b = jnp.pad(_row(params['fc4']['b']), ((0, 0), (0, 7)))
    res = _tc_call(
        _final_body, (NGRAPH, 8), *reps, batch.reshape(1, N_NODES),
        params['fc1']['W'], _row(params['fc1']['b']),
        params['fc2']['W'], _row(params['fc2']['b']),
        params['fc3']['W'], _row(params['fc3']['b']),
        f4W, f4b)
    return res[:, 0]


# speedup vs baseline: 14.9044x; 1.5004x over previous
import functools

import jax
import jax.numpy as jnp
from jax import lax
from jax.experimental import pallas as pl
from jax.experimental.pallas import tpu as pltpu
from jax.experimental.pallas import tpu_sc as plsc

N_NODES = 10000
E_EDGES = 160000
DIM = 64
NGRAPH = 64

NC = 2
NS = 16
NW = NC * NS

EPT = E_EDGES // NW
CMIN = 125
NCH = EPT // CMIN
NB = 8
NPAD = NW * 4 * 80

_mesh = plsc.VectorSubcoreMesh(
    core_axis_name="c", subcore_axis_name="s", num_cores=NC, num_subcores=NS)


@functools.partial(
    pl.kernel,
    out_type=jax.ShapeDtypeStruct((3, NW, 4, 80, DIM), jnp.float32),
    mesh=_mesh,
    compiler_params=pltpu.CompilerParams(use_tc_tiling_on_sc=False),
    scratch_types=[
        pltpu.VMEM((4, 80), jnp.int32),
        pltpu.VMEM((80, DIM), jnp.float32),
        pltpu.SemaphoreType.DMA,
    ],
)
def _gather_nodes(x_hbm, w1_hbm, w2_hbm, w3_hbm, out_hbm, xs_v, rows_v, sem):
    c = lax.axis_index("c")
    s = lax.axis_index("s")
    wid = s * NC + c
    pltpu.sync_copy(x_hbm.at[wid], xs_v)
    for j, w in enumerate((w1_hbm, w2_hbm, w3_hbm)):
        for k in range(4):
            pltpu.async_copy(w.at[xs_v.at[k]], rows_v, sem).wait()
            pltpu.sync_copy(rows_v, out_hbm.at[j, wid, k])


NGRP = NCH // NB


@functools.partial(
    pl.kernel,
    out_type=jax.ShapeDtypeStruct((NC, N_NODES, DIM), jnp.float32),
    mesh=_mesh,
    compiler_params=pltpu.CompilerParams(use_tc_tiling_on_sc=False),
    scratch_types=[
        pltpu.VMEM((NCH, CMIN), jnp.int32),
        pltpu.VMEM((NCH, CMIN), jnp.int32),
        [pltpu.VMEM((CMIN, DIM), jnp.float32) for _ in range(NB)],
        pltpu.VMEM((CMIN, DIM), jnp.float32),
        pltpu.VMEM_SHARED((N_NODES, DIM), jnp.float32),
        [pltpu.SemaphoreType.DMA for _ in range(NB)],
        [pltpu.SemaphoreType.DMA for _ in range(NB)],
    ],
)
def _agg(tab_hbm, src_hbm, dst_hbm, out_hbm, src_v, dst_v, rows, zbuf_v,
         acc_sh, semg, sems):
    c = lax.axis_index("c")
    s = lax.axis_index("s")
    wid = s * NC + c
    rps = N_NODES // NS

    zero = jnp.zeros((16,), jnp.float32)

    def _zero_row(i, carry):
        for cc in range(DIM // 16):
            zbuf_v[i, pl.ds(cc * 16, 16)] = zero
        return carry

    lax.fori_loop(0, CMIN, _zero_row, 0)
    base = s * rps
    for k in range(rps // CMIN):
        pltpu.sync_copy(zbuf_v, acc_sh.at[pl.ds(base + k * CMIN, CMIN)])

    pltpu.sync_copy(src_hbm.at[wid], src_v)
    pltpu.sync_copy(dst_hbm.at[wid], dst_v)
    plsc.subcore_barrier()

    def _fire_g(ch, b):
        pltpu.async_copy(tab_hbm.at[src_v.at[ch]], rows[b], semg[b])

    def _wait_g(b):
        pltpu.make_async_copy(
            tab_hbm.at[src_v.at[0]], rows[b], semg[b]).wait()

    def _fire_s(ch, b):
        pltpu.async_copy(rows[b], acc_sh.at[dst_v.at[ch]], sems[b], add=True)

    def _wait_s(b):
        pltpu.make_async_copy(rows[b], acc_sh.at[dst_v.at[0]], sems[b]).wait()

    for b in range(NB):
        _fire_g(b, b)

    def _group(g, carry):
        for b in range(NB):
            _wait_g(b)
            _fire_s(g * NB + b, b)
        for b in range(NB):
            _wait_s(b)
            _fire_g((g + 1) * NB + b, b)
        return carry

    lax.fori_loop(0, NGRP - 1, _group, 0)
    for b in range(NB):
        _wait_g(b)
        _fire_s((NGRP - 1) * NB + b, b)
    for b in range(NB):
        _wait_s(b)

    plsc.subcore_barrier()
    pltpu.sync_copy(acc_sh.at[pl.ds(base, rps)],
                    out_hbm.at[c, pl.ds(base, rps)])


def _dot(a, b):
    return jnp.dot(a, b, preferred_element_type=jnp.float32)


def _bn_tail(cat, mW1, mb1, mW2, mb2, gam, bet, out):
    h = jnp.maximum(_dot(cat, mW1[...]) + mb1[...], 0.0)
    h = _dot(h, mW2[...]) + mb2[...]
    m = jnp.mean(h, axis=0, keepdims=True)
    v = jnp.mean((h - m) * (h - m), axis=0, keepdims=True)
    out[...] = gam[...] * (h - m) * lax.rsqrt(v + 1e-5) + bet[...]


def _layer1_body(g1, g2, g3, pa, pb, pc,
                 e1, b11, W21, b21, e2, b12, W22, b22, e3, b13, W23, b23,
                 mW1, mb1, mW2, mb2, gam, bet, out):
    xs = []
    for g, pp, eps, b1, W2, b2 in (
            (g1, pa, e1, b11, W21, b21),
            (g2, pb, e2, b12, W22, b22),
            (g3, pc, e3, b13, W23, b23)):
        t = jnp.maximum(
            (1.0 + eps[0, 0]) * g[...] + pp[0] + pp[1] + b1[...], 0.0)
        xs.append(jnp.maximum(_dot(t, W2[...]) + b2[...], 0.0))
    _bn_tail(jnp.concatenate(xs, axis=1), mW1, mb1, mW2, mb2, gam, bet, out)


def _layer_body(h, pa, pb, pc,
                e1, W11, b11, W21, b21, e2, W12, b12, W22, b22,
                e3, W13, b13, W23, b23,
                mW1, mb1, mW2, mb2, gam, bet, out):
    xs = []
    for pp, eps, W1, b1, W2, b2 in (
            (pa, e1, W11, b11, W21, b21),
            (pb, e2, W12, b12, W22, b22),
            (pc, e3, W13, b13, W23, b23)):
        hin = (1.0 + eps[0, 0]) * h[...] + pp[0] + pp[1]
        t = jnp.maximum(_dot(hin, W1[...]) + b1[...], 0.0)
        xs.append(jnp.maximum(_dot(t, W2[...]) + b2[...], 0.0))
    _bn_tail(jnp.concatenate(xs, axis=1), mW1, mb1, mW2, mb2, gam, bet, out)


def _final_body(r1, r2, r3, r4, bt,
                f1W, f1b, f2W, f2b, f3W, f3b, f4W, f4b, out):
    sel = lax.broadcasted_iota(jnp.int32, (NGRAPH, N_NODES), 0)
    P = (sel == bt[...]).astype(jnp.float32)
    counts = jnp.sum(P, axis=1, keepdims=True)
    hcat = jnp.concatenate([r1[...], r2[...], r3[...], r4[...]], axis=1)
    pooled = _dot(P, hcat) / jnp.maximum(counts, 1.0)
    h = jnp.maximum(_dot(pooled, f1W[...]) + f1b[...], 0.0)
    h = jnp.maximum(_dot(h, f2W[...]) + f2b[...], 0.0)
    h = jnp.maximum(_dot(h, f3W[...]) + f3b[...], 0.0)
    out[...] = _dot(h, f4W[...]) + f4b[...]


def _tc_call(body, out_shape, *args):
    return pl.pallas_call(
        body, out_shape=jax.ShapeDtypeStruct(out_shape, jnp.float32))(*args)


def _row(v):
    return v.reshape(1, -1)


def kernel(x, edge_index_1, edge_index_2, edge_index_3, batch, params):
    x_pad = jnp.concatenate(
        [x, jnp.zeros((NPAD - N_NODES,), jnp.int32)]).reshape(NW, 4, 80)
    srcs, dsts = [], []
    for e in (edge_index_1, edge_index_2, edge_index_3):
        srcs.append(e[0].reshape(NW, NCH, CMIN))
        dsts.append(e[1].reshape(NW, NCH, CMIN))

    g = _gather_nodes(x_pad, params['conv1_1']['W1'], params['conv1_2']['W1'],
                      params['conv1_3']['W1'])
    g = g.reshape(3, NPAD, DIM)[:, :N_NODES]
    gs = [g[0], g[1], g[2]]

    parts = [_agg(gs[j], srcs[j], dsts[j]) for j in range(3)]
    l1args = []
    for j in range(3):
        q = params['conv1_%d' % (j + 1)]
        l1args += [q['eps'].reshape(1, 1), _row(q['b1']), q['W2'],
                   _row(q['b2'])]
    q = params['mlp_1']
    bnq = params['bn_1']
    h = _tc_call(_layer1_body, (N_NODES, DIM), *gs, *parts, *l1args,
                 q['W1'], _row(q['b1']), q['W2'], _row(q['b2']),
                 _row(bnq['gamma']), _row(bnq['beta']))
    reps = [h]

    for l in range(2, 5):
        parts = [_agg(h, srcs[j], dsts[j]) for j in range(3)]
        largs = []
        for j in range(3):
            q = params['conv%d_%d' % (l, j + 1)]
            largs += [q['eps'].reshape(1, 1), q['W1'], _row(q['b1']),
                      q['W2'], _row(q['b2'])]
        q = params['mlp_%d' % l]
        bnq = params['bn_%d' % l]
        h = _tc_call(_layer_body, (N_NODES, DIM), h, *parts, *largs,
                     q['W1'], _row(q['b1']), q['W2'], _row(q['b2']),
                     _row(bnq['gamma']), _row(bnq['beta']))
        reps.append(h)

    f4W = jnp.pad(params['fc4']['W'], ((0, 0), (0, 7)))
    f4b = jnp.pad(_row(params['fc4']['b']), ((0, 0), (0, 7)))
    res = _tc_call(
        _final_body, (NGRAPH, 8), *reps, batch.reshape(1, N_NODES),
        params['fc1']['W'], _row(params['fc1']['b']),
        params['fc2']['W'], _row(params['fc2']['b']),
        params['fc3']['W'], _row(params['fc3']['b']),
        f4W, f4b)
    return res[:, 0]
